# Initial kernel scaffold; baseline (speedup 1.0000x reference)
import functools

import jax
import jax.numpy as jnp
from jax import lax
from jax.experimental import pallas as pl
from jax.experimental.pallas import tpu as pltpu
from jax.experimental.pallas import tpu_sc as plsc

BN = 10000
E = 320000
H = 4
DH = 32
OUT = 128

NC = 2
NS = 16
NW = NC * NS
EPW = E // NW
C = 80
NCHUNK = EPW // C
GROUPS = C // 16
ROWS_PER_TILE = BN // NS


def _pre_body(x_ref, emb_ref, wq_ref, wk_ref, wv_ref, aqf_ref, akf_ref,
              sqk_ref, v_ref, eb_ref):
    jrow = lax.broadcasted_iota(jnp.int32, (128, 8), 0) // DH
    hcol = lax.broadcasted_iota(jnp.int32, (128, 8), 1)
    aq = aqf_ref[...]
    ak = akf_ref[...]
    aq_m = jnp.where(jrow == hcol, aq, 0.0)
    ak_m = jnp.where(jrow == hcol - 4, ak, 0.0)
    wqk = (jnp.dot(wq_ref[...], aq_m, preferred_element_type=jnp.float32)
           + jnp.dot(wk_ref[...], ak_m, preferred_element_type=jnp.float32))
    emb = emb_ref[...]
    sqk = jnp.dot(emb, wqk, preferred_element_type=jnp.float32)
    sqk_ref[...] = sqk
    v_ref[...] = (
        jnp.dot(x_ref[...], wv_ref[:128, :], preferred_element_type=jnp.float32)
        + jnp.dot(emb, wv_ref[128:, :], preferred_element_type=jnp.float32))
    mx = jnp.max(sqk, axis=0, keepdims=True)
    eb = mx[:, :4] + mx[:, 4:]
    eb_ref[...] = jnp.where(eb >= 0, eb, 0.2 * eb)


_pre_call = pl.pallas_call(
    _pre_body,
    out_shape=[
        jax.ShapeDtypeStruct((BN, 8), jnp.float32),
        jax.ShapeDtypeStruct((BN, OUT), jnp.float32),
        jax.ShapeDtypeStruct((1, 4), jnp.float32),
    ],
)


def _sc_body(sqk_hbm, src_hbm, dst_hbm, v_hbm, eb_hbm,
             zout_hbm, sout_hbm,
             sqk_v, vchunk, eexp, srcv, dstv, ebv, z_sh, s_sh, sem):
    c = lax.axis_index("c")
    s = lax.axis_index("s")
    wid = c * NS + s
    iota16 = lax.iota(jnp.int32, 16)
    zeros16 = jnp.zeros((16,), jnp.float32)

    def _zrow(i, _):
        for j in range(8):
            vchunk[i, pl.ds(j * 16, 16)] = zeros16
        return 0
    lax.fori_loop(0, 128, _zrow, 0)

    def _zrow2(i, _):
        eexp[i, pl.ds(0, 16)] = zeros16
        return 0
    lax.fori_loop(0, C, _zrow2, 0)

    r0 = s * ROWS_PER_TILE
    for k in range(4):
        pltpu.sync_copy(vchunk, z_sh.at[pl.ds(r0 + k * 128, 128)])
    pltpu.sync_copy(vchunk.at[pl.ds(0, 113)],
                    z_sh.at[pl.ds(r0 + 512, 113)])
    for k in range(7):
        pltpu.sync_copy(eexp, s_sh.at[pl.ds(r0 + k * 80, 80)])
    pltpu.sync_copy(eexp.at[pl.ds(0, 65)], s_sh.at[pl.ds(r0 + 560, 65)])

    pltpu.sync_copy(sqk_hbm, sqk_v)
    pltpu.sync_copy(eb_hbm, ebv)
    ebh = [plsc.load_gather(ebv, [jnp.full((16,), h, jnp.int32)])
           for h in range(H)]

    plsc.subcore_barrier()

    def _chunk(ci, _):
        base = wid * EPW + ci * C
        pltpu.sync_copy(src_hbm.at[pl.ds(base, C)], srcv)
        pltpu.sync_copy(dst_hbm.at[pl.ds(base, C)], dstv)
        gat = pltpu.async_copy(v_hbm.at[srcv], vchunk.at[pl.ds(0, C)], sem)

        for g in range(GROUPS):
            dv = dstv[pl.ds(g * 16, 16)]
            sv = srcv[pl.ds(g * 16, 16)]
            dq = dv * 8
            sk8 = sv * 8
            rows = iota16 + g * 16
            for h in range(H):
                qq = plsc.load_gather(sqk_v, [dq + h])
                kk = plsc.load_gather(sqk_v, [sk8 + (4 + h)])
                e = qq + kk
                e = jnp.where(e >= 0, e, 0.2 * e)
                w = jnp.exp(e - ebh[h])
                plsc.store_scatter(eexp, [rows, jnp.full((16,), h, jnp.int32)], w)

        pltpu.sync_copy(eexp, s_sh.at[dstv], add=True)

        gat.wait()

        def _scale(ei, _):
            for h in range(H):
                w = plsc.load_gather(
                    eexp, [jnp.full((16,), ei, jnp.int32),
                           jnp.full((16,), h, jnp.int32)])
                for j in (2 * h, 2 * h + 1):
                    v = vchunk[ei, pl.ds(j * 16, 16)]
                    vchunk[ei, pl.ds(j * 16, 16)] = v * w
            return 0
        lax.fori_loop(0, C, _scale, 0)

        pltpu.sync_copy(vchunk.at[pl.ds(0, C)], z_sh.at[dstv], add=True)
        return 0

    lax.fori_loop(0, NCHUNK, _chunk, 0)

    plsc.subcore_barrier()
    pltpu.sync_copy(z_sh.at[pl.ds(r0, ROWS_PER_TILE)],
                    zout_hbm.at[c, pl.ds(r0, ROWS_PER_TILE)])
    pltpu.sync_copy(s_sh.at[pl.ds(r0, ROWS_PER_TILE)],
                    sout_hbm.at[c, pl.ds(r0, ROWS_PER_TILE)])


_sc_call = functools.partial(
    pl.kernel,
    out_type=[
        jax.ShapeDtypeStruct((NC, BN, OUT), jnp.float32),
        jax.ShapeDtypeStruct((NC, BN, 16), jnp.float32),
    ],
    mesh=plsc.VectorSubcoreMesh(core_axis_name="c", subcore_axis_name="s"),
    scratch_types=[
        pltpu.VMEM((BN * 8,), jnp.float32),
        pltpu.VMEM((128, OUT), jnp.float32),
        pltpu.VMEM((C, 16), jnp.float32),
        pltpu.VMEM((C,), jnp.int32),
        pltpu.VMEM((C,), jnp.int32),
        pltpu.VMEM((16,), jnp.float32),
        pltpu.VMEM_SHARED((BN, OUT), jnp.float32),
        pltpu.VMEM_SHARED((BN, 16), jnp.float32),
        pltpu.SemaphoreType.DMA,
    ],
)(_sc_body)


def _post_body(zp_ref, sp_ref, g_ref, b_ref, out_ref):
    z = zp_ref[0] + zp_ref[1]
    ssum = sp_ref[0] + sp_ref[1]
    rec = 1.0 / (ssum + 1e-8)
    trow = lax.broadcasted_iota(jnp.int32, (16, OUT), 0)
    jcol = lax.broadcasted_iota(jnp.int32, (16, OUT), 1) // DH
    p = jnp.where(trow == jcol, 1.0, 0.0)
    z = z * jnp.dot(rec, p, preferred_element_type=jnp.float32)
    z = jnp.where(z > 0, z, jnp.exp(jnp.minimum(z, 0.0)) - 1.0)
    mean = jnp.mean(z, axis=-1, keepdims=True)
    var = jnp.mean((z - mean) ** 2, axis=-1, keepdims=True)
    zn = (z - mean) * lax.rsqrt(var + 1e-5)
    out_ref[...] = zn * g_ref[...] + b_ref[...]


_post_call = pl.pallas_call(
    _post_body,
    out_shape=jax.ShapeDtypeStruct((BN, OUT), jnp.float32),
)


def kernel(x, emb, edge_index, Wq, Wk, Wv, attn_vec, gamma, beta):
    aqf = attn_vec[:, :DH].reshape(128, 1)
    akf = attn_vec[:, DH:].reshape(128, 1)
    sqk, v, eb = _pre_call(x, emb, Wq, Wk, Wv, aqf, akf)
    eb16 = jnp.pad(eb.reshape(4), (0, 12))
    zp, sp = _sc_call(sqk.reshape(BN * 8), edge_index[0], edge_index[1],
                      v, eb16)
    return _post_call(zp, sp, gamma.reshape(1, OUT), beta.reshape(1, OUT))


# speedup vs baseline: 51.2483x; 51.2483x over previous
import functools

import jax
import jax.numpy as jnp
from jax import lax
from jax.experimental import pallas as pl
from jax.experimental.pallas import tpu as pltpu
from jax.experimental.pallas import tpu_sc as plsc

BN = 10000
E = 320000
H = 4
DH = 32
OUT = 128

NC = 2
NS = 16
NW = NC * NS
EPW = E // NW
C = 80
NCHUNK = EPW // C
GROUPS = C // 16
ROWS_PER_TILE = 624


BLK = 1000


def _pre_body(x_ref, emb_ref, wq_ref, wk_ref, wv_ref, aqf_ref, akf_ref,
              sqk_ref, v_ref, eb_ref, mxs_ref):
    i = pl.program_id(0)
    jrow = lax.broadcasted_iota(jnp.int32, (128, 16), 0) // DH
    hcol = lax.broadcasted_iota(jnp.int32, (128, 16), 1)
    aq = aqf_ref[...]
    ak = akf_ref[...]
    aq_m = jnp.where(jrow == hcol, aq, 0.0)
    ak_m = jnp.where(jrow == hcol - 4, ak, 0.0)
    wqk = (jnp.dot(wq_ref[...], aq_m, preferred_element_type=jnp.float32)
           + jnp.dot(wk_ref[...], ak_m, preferred_element_type=jnp.float32))
    emb = emb_ref[...]
    sqk = jnp.dot(emb, wqk, preferred_element_type=jnp.float32)
    sqk_ref[...] = sqk
    v_ref[...] = (
        jnp.dot(x_ref[...], wv_ref[:128, :], preferred_element_type=jnp.float32)
        + jnp.dot(emb, wv_ref[128:, :], preferred_element_type=jnp.float32))
    mx = jnp.max(sqk, axis=0, keepdims=True)

    @pl.when(i == 0)
    def _init():
        mxs_ref[...] = mx

    @pl.when(i > 0)
    def _acc():
        mxs_ref[...] = jnp.maximum(mxs_ref[...], mx)

    @pl.when(i == BN // BLK - 1)
    def _fin():
        m = mxs_ref[...]
        eb = m[:, :4] + m[:, 4:8]
        eb_ref[...] = jnp.where(eb >= 0, eb, 0.2 * eb)


_pre_call = pl.pallas_call(
    _pre_body,
    grid=(BN // BLK,),
    in_specs=[
        pl.BlockSpec((BLK, 128), lambda i: (i, 0)),
        pl.BlockSpec((BLK, 128), lambda i: (i, 0)),
        pl.BlockSpec((128, 128), lambda i: (0, 0)),
        pl.BlockSpec((128, 128), lambda i: (0, 0)),
        pl.BlockSpec((256, 128), lambda i: (0, 0)),
        pl.BlockSpec((128, 1), lambda i: (0, 0)),
        pl.BlockSpec((128, 1), lambda i: (0, 0)),
    ],
    out_specs=[
        pl.BlockSpec((BLK, 16), lambda i: (i, 0)),
        pl.BlockSpec((BLK, OUT), lambda i: (i, 0)),
        pl.BlockSpec((1, 4), lambda i: (0, 0)),
    ],
    out_shape=[
        jax.ShapeDtypeStruct((BN, 16), jnp.float32),
        jax.ShapeDtypeStruct((BN, OUT), jnp.float32),
        jax.ShapeDtypeStruct((1, 4), jnp.float32),
    ],
    scratch_shapes=[pltpu.VMEM((1, 16), jnp.float32)],
)


def _sc_body(sqk_hbm, src_hbm, dst_hbm, v_hbm, eb_hbm,
             zout_hbm, sout_hbm,
             vchunk, eexp, sqd, sks, srcv, dstv, ebv,
             z_sh, s_sh, sem, sem2):
    c = lax.axis_index("c")
    s = lax.axis_index("s")
    wid = c * NS + s
    iota16 = lax.iota(jnp.int32, 16)
    zeros16 = jnp.zeros((16,), jnp.float32)

    def _zrow(i, _):
        for j in range(8):
            vchunk[i, pl.ds(j * 16, 16)] = zeros16
        return 0
    lax.fori_loop(0, C, _zrow, 0)

    zeros16f = jnp.zeros((16,), jnp.float32)
    for col in range(8):
        for gi in range(GROUPS):
            plsc.store_scatter(
                eexp, [iota16 + gi * 16, jnp.full((16,), col, jnp.int32)],
                zeros16f)

    r0 = s * ROWS_PER_TILE
    for k in range(7):
        pltpu.sync_copy(vchunk, z_sh.at[pl.ds(r0 + k * 80, 80)])
        pltpu.sync_copy(eexp, s_sh.at[pl.ds(r0 + k * 80, 80)])
    pltpu.sync_copy(vchunk.at[pl.ds(0, 64)], z_sh.at[pl.ds(r0 + 560, 64)])
    pltpu.sync_copy(eexp.at[pl.ds(0, 64)], s_sh.at[pl.ds(r0 + 560, 64)])

    @pl.when(s == NS - 1)
    def _zero_tail():
        pltpu.sync_copy(vchunk.at[pl.ds(0, 16)],
                        z_sh.at[pl.ds(NS * ROWS_PER_TILE, 16)])
        pltpu.sync_copy(eexp.at[pl.ds(0, 16)],
                        s_sh.at[pl.ds(NS * ROWS_PER_TILE, 16)])

    pltpu.sync_copy(eb_hbm, ebv)
    ebh = [ebv[pl.ds(h * 16, 16)] for h in range(H)]

    plsc.subcore_barrier()

    def _chunk(ci, _):
        base = wid * EPW + ci * C
        pltpu.sync_copy(src_hbm.at[pl.ds(base, C)], srcv)
        pltpu.sync_copy(dst_hbm.at[pl.ds(base, C)], dstv)
        gat = pltpu.async_copy(v_hbm.at[srcv], vchunk, sem)
        gq = pltpu.async_copy(sqk_hbm.at[dstv], sqd, sem2)
        pltpu.sync_copy(sqk_hbm.at[srcv], sks)
        gq.wait()

        for g in range(GROUPS):
            rows = iota16 + g * 16
            for h in range(H):
                qq = plsc.load_gather(
                    sqd, [rows, jnp.full((16,), h, jnp.int32)])
                kk = plsc.load_gather(
                    sks, [rows, jnp.full((16,), 4 + h, jnp.int32)])
                e = qq + kk
                e = jnp.where(e >= 0, e, 0.2 * e)
                w = jnp.exp(e - ebh[h])
                plsc.store_scatter(
                    eexp, [rows, jnp.full((16,), 4 + h, jnp.int32)], w)

        pltpu.sync_copy(eexp, s_sh.at[dstv], add=True)

        gat.wait()

        def _scale(ei, _):
            for h in range(H):
                w = plsc.load_gather(
                    eexp, [jnp.full((16,), ei, jnp.int32),
                           jnp.full((16,), 4 + h, jnp.int32)])
                for j in (2 * h, 2 * h + 1):
                    v = vchunk[ei, pl.ds(j * 16, 16)]
                    vchunk[ei, pl.ds(j * 16, 16)] = v * w
            return 0
        lax.fori_loop(0, C, _scale, 0)

        pltpu.sync_copy(vchunk, z_sh.at[dstv], add=True)
        return 0

    lax.fori_loop(0, NCHUNK, _chunk, 0)

    plsc.subcore_barrier()
    pltpu.sync_copy(z_sh.at[pl.ds(r0, ROWS_PER_TILE)],
                    zout_hbm.at[c, pl.ds(r0, ROWS_PER_TILE)])
    pltpu.sync_copy(s_sh.at[pl.ds(r0, ROWS_PER_TILE)],
                    sout_hbm.at[c, pl.ds(r0, ROWS_PER_TILE)])

    @pl.when(s == NS - 1)
    def _drain_tail():
        t0 = NS * ROWS_PER_TILE
        pltpu.sync_copy(z_sh.at[pl.ds(t0, 16)],
                        zout_hbm.at[c, pl.ds(t0, 16)])
        pltpu.sync_copy(s_sh.at[pl.ds(t0, 16)],
                        sout_hbm.at[c, pl.ds(t0, 16)])


_sc_call = functools.partial(
    pl.kernel,
    out_type=[
        jax.ShapeDtypeStruct((NC, BN, OUT), jnp.float32),
        jax.ShapeDtypeStruct((NC, BN, 8), jnp.float32),
    ],
    mesh=plsc.VectorSubcoreMesh(core_axis_name="c", subcore_axis_name="s"),
    compiler_params=pltpu.CompilerParams(needs_layout_passes=False,
                                         use_tc_tiling_on_sc=False),
    scratch_types=[
        pltpu.VMEM((C, OUT), jnp.float32),
        pltpu.VMEM((C, 8), jnp.float32),
        pltpu.VMEM((C, 16), jnp.float32),
        pltpu.VMEM((C, 16), jnp.float32),
        pltpu.VMEM((C,), jnp.int32),
        pltpu.VMEM((C,), jnp.int32),
        pltpu.VMEM((64,), jnp.float32),
        pltpu.VMEM_SHARED((BN, OUT), jnp.float32),
        pltpu.VMEM_SHARED((BN, 8), jnp.float32),
        pltpu.SemaphoreType.DMA,
        pltpu.SemaphoreType.DMA,
    ],
)(_sc_body)


def _post_body(zp_ref, sp_ref, g_ref, b_ref, out_ref):
    z = zp_ref[0] + zp_ref[1]
    ssum = sp_ref[0] + sp_ref[1]
    rec = 1.0 / (ssum + 1e-8)
    trow = lax.broadcasted_iota(jnp.int32, (8, OUT), 0) - 4
    jcol = lax.broadcasted_iota(jnp.int32, (8, OUT), 1) // DH
    p = jnp.where(trow == jcol, 1.0, 0.0)
    z = z * jnp.dot(rec, p, preferred_element_type=jnp.float32)
    z = jnp.where(z > 0, z, jnp.exp(jnp.minimum(z, 0.0)) - 1.0)
    mean = jnp.mean(z, axis=-1, keepdims=True)
    var = jnp.mean((z - mean) ** 2, axis=-1, keepdims=True)
    zn = (z - mean) * lax.rsqrt(var + 1e-5)
    out_ref[...] = zn * g_ref[...] + b_ref[...]


_post_call = pl.pallas_call(
    _post_body,
    grid=(BN // BLK,),
    in_specs=[
        pl.BlockSpec((NC, BLK, OUT), lambda i: (0, i, 0)),
        pl.BlockSpec((NC, BLK, 8), lambda i: (0, i, 0)),
        pl.BlockSpec((1, OUT), lambda i: (0, 0)),
        pl.BlockSpec((1, OUT), lambda i: (0, 0)),
    ],
    out_specs=pl.BlockSpec((BLK, OUT), lambda i: (i, 0)),
    out_shape=jax.ShapeDtypeStruct((BN, OUT), jnp.float32),
)


def kernel(x, emb, edge_index, Wq, Wk, Wv, attn_vec, gamma, beta):
    aqf = attn_vec[:, :DH].reshape(128, 1)
    akf = attn_vec[:, DH:].reshape(128, 1)
    sqk, v, eb = _pre_call(x, emb, Wq, Wk, Wv, aqf, akf)
    eb64 = jnp.broadcast_to(eb.reshape(4, 1), (4, 16)).reshape(64)
    zp, sp = _sc_call(sqk, edge_index[0], edge_index[1], v, eb64)
    return _post_call(zp, sp, gamma.reshape(1, OUT), beta.reshape(1, OUT))


# speedup vs baseline: 52.0906x; 1.0164x over previous
import functools

import jax
import jax.numpy as jnp
from jax import lax
from jax.experimental import pallas as pl
from jax.experimental.pallas import tpu as pltpu
from jax.experimental.pallas import tpu_sc as plsc

BN = 10000
E = 320000
H = 4
DH = 32
OUT = 128

NC = 2
NS = 16
NW = NC * NS
EPW = E // NW
C = 80
NCHUNK = EPW // C
GROUPS = C // 16
ROWS_PER_TILE = 624


BLK = 1000


def _pre_body(x_ref, emb_ref, wq_ref, wk_ref, wv_ref, aqf_ref, akf_ref,
              sqk_ref, v_ref, eb_ref, mxs_ref):
    i = pl.program_id(0)
    jrow = lax.broadcasted_iota(jnp.int32, (128, 16), 0) // DH
    hcol = lax.broadcasted_iota(jnp.int32, (128, 16), 1)
    aq = aqf_ref[...]
    ak = akf_ref[...]
    aq_m = jnp.where(jrow == hcol, aq, 0.0)
    ak_m = jnp.where(jrow == hcol - 4, ak, 0.0)
    wqk = (jnp.dot(wq_ref[...], aq_m, preferred_element_type=jnp.float32)
           + jnp.dot(wk_ref[...], ak_m, preferred_element_type=jnp.float32))
    emb = emb_ref[...]
    sqk = jnp.dot(emb, wqk, preferred_element_type=jnp.float32)
    sqk_ref[...] = sqk
    v_ref[...] = (
        jnp.dot(x_ref[...], wv_ref[:128, :], preferred_element_type=jnp.float32)
        + jnp.dot(emb, wv_ref[128:, :], preferred_element_type=jnp.float32))
    mx = jnp.max(sqk, axis=0, keepdims=True)

    @pl.when(i == 0)
    def _init():
        mxs_ref[...] = mx

    @pl.when(i > 0)
    def _acc():
        mxs_ref[...] = jnp.maximum(mxs_ref[...], mx)

    @pl.when(i == BN // BLK - 1)
    def _fin():
        m = mxs_ref[...]
        eb = m[:, :4] + m[:, 4:8]
        eb_ref[...] = jnp.where(eb >= 0, eb, 0.2 * eb)


_pre_call = pl.pallas_call(
    _pre_body,
    grid=(BN // BLK,),
    in_specs=[
        pl.BlockSpec((BLK, 128), lambda i: (i, 0)),
        pl.BlockSpec((BLK, 128), lambda i: (i, 0)),
        pl.BlockSpec((128, 128), lambda i: (0, 0)),
        pl.BlockSpec((128, 128), lambda i: (0, 0)),
        pl.BlockSpec((256, 128), lambda i: (0, 0)),
        pl.BlockSpec((128, 1), lambda i: (0, 0)),
        pl.BlockSpec((128, 1), lambda i: (0, 0)),
    ],
    out_specs=[
        pl.BlockSpec((BLK, 16), lambda i: (i, 0)),
        pl.BlockSpec((BLK, OUT), lambda i: (i, 0)),
        pl.BlockSpec((1, 4), lambda i: (0, 0)),
    ],
    out_shape=[
        jax.ShapeDtypeStruct((BN, 16), jnp.float32),
        jax.ShapeDtypeStruct((BN, OUT), jnp.float32),
        jax.ShapeDtypeStruct((1, 4), jnp.float32),
    ],
    scratch_shapes=[pltpu.VMEM((1, 16), jnp.float32)],
)


def _sc_body(sqk_hbm, src_hbm, dst_hbm, v_hbm, eb_hbm,
             zout_hbm, sout_hbm,
             vchunk, eexp, sqd, sks, srcv, dstv, dstz, ebv,
             z_sh, s_sh, sem, sem2, sem3, sems, semz):
    c = lax.axis_index("c")
    s = lax.axis_index("s")
    wid = c * NS + s
    iota16 = lax.iota(jnp.int32, 16)
    zeros16 = jnp.zeros((16,), jnp.float32)

    def _zrow(i, _):
        for j in range(8):
            vchunk[i, pl.ds(j * 16, 16)] = zeros16
        return 0
    lax.fori_loop(0, C, _zrow, 0)

    zeros16f = jnp.zeros((16,), jnp.float32)
    for col in range(8):
        for gi in range(GROUPS):
            plsc.store_scatter(
                eexp, [iota16 + gi * 16, jnp.full((16,), col, jnp.int32)],
                zeros16f)

    r0 = s * ROWS_PER_TILE
    for k in range(7):
        pltpu.sync_copy(vchunk, z_sh.at[pl.ds(r0 + k * 80, 80)])
        pltpu.sync_copy(eexp, s_sh.at[pl.ds(r0 + k * 80, 80)])
    pltpu.sync_copy(vchunk.at[pl.ds(0, 64)], z_sh.at[pl.ds(r0 + 560, 64)])
    pltpu.sync_copy(eexp.at[pl.ds(0, 64)], s_sh.at[pl.ds(r0 + 560, 64)])

    @pl.when(s == NS - 1)
    def _zero_tail():
        pltpu.sync_copy(vchunk.at[pl.ds(0, 16)],
                        z_sh.at[pl.ds(NS * ROWS_PER_TILE, 16)])
        pltpu.sync_copy(eexp.at[pl.ds(0, 16)],
                        s_sh.at[pl.ds(NS * ROWS_PER_TILE, 16)])

    pltpu.sync_copy(eb_hbm, ebv)
    ebh = [ebv[pl.ds(h * 16, 16)] for h in range(H)]

    plsc.subcore_barrier()

    def _chunk(ci, _):
        base = wid * EPW + ci * C
        pltpu.sync_copy(src_hbm.at[pl.ds(base, C)], srcv)
        pltpu.sync_copy(dst_hbm.at[pl.ds(base, C)], dstv)
        gat = pltpu.async_copy(v_hbm.at[srcv], vchunk, sem)
        gq = pltpu.async_copy(sqk_hbm.at[dstv], sqd, sem2)
        gk = pltpu.async_copy(sqk_hbm.at[srcv], sks, sem3)
        gq.wait()
        gk.wait()

        for g in range(GROUPS):
            rows = iota16 + g * 16
            for h in range(H):
                qq = plsc.load_gather(
                    sqd, [rows, jnp.full((16,), h, jnp.int32)])
                kk = plsc.load_gather(
                    sks, [rows, jnp.full((16,), 4 + h, jnp.int32)])
                e = qq + kk
                e = jnp.where(e >= 0, e, 0.2 * e)
                w = jnp.exp(e - ebh[h])
                plsc.store_scatter(
                    eexp, [rows, jnp.full((16,), 4 + h, jnp.int32)], w)

        gs = pltpu.async_copy(eexp, s_sh.at[dstv], sems, add=True)

        gat.wait()

        def _scale(ei, _):
            for h in range(H):
                w = plsc.load_gather(
                    eexp, [jnp.full((16,), ei, jnp.int32),
                           jnp.full((16,), 4 + h, jnp.int32)])
                for j in (2 * h, 2 * h + 1):
                    v = vchunk[ei, pl.ds(j * 16, 16)]
                    vchunk[ei, pl.ds(j * 16, 16)] = v * w
            return 0
        lax.fori_loop(0, C, _scale, 0)

        gs.wait()
        pltpu.sync_copy(vchunk, z_sh.at[dstv], add=True)
        return 0

    lax.fori_loop(0, NCHUNK, _chunk, 0)

    plsc.subcore_barrier()
    pltpu.sync_copy(z_sh.at[pl.ds(r0, ROWS_PER_TILE)],
                    zout_hbm.at[c, pl.ds(r0, ROWS_PER_TILE)])
    pltpu.sync_copy(s_sh.at[pl.ds(r0, ROWS_PER_TILE)],
                    sout_hbm.at[c, pl.ds(r0, ROWS_PER_TILE)])

    @pl.when(s == NS - 1)
    def _drain_tail():
        t0 = NS * ROWS_PER_TILE
        pltpu.sync_copy(z_sh.at[pl.ds(t0, 16)],
                        zout_hbm.at[c, pl.ds(t0, 16)])
        pltpu.sync_copy(s_sh.at[pl.ds(t0, 16)],
                        sout_hbm.at[c, pl.ds(t0, 16)])


_sc_call = functools.partial(
    pl.kernel,
    out_type=[
        jax.ShapeDtypeStruct((NC, BN, OUT), jnp.float32),
        jax.ShapeDtypeStruct((NC, BN, 8), jnp.float32),
    ],
    mesh=plsc.VectorSubcoreMesh(core_axis_name="c", subcore_axis_name="s"),
    compiler_params=pltpu.CompilerParams(needs_layout_passes=False,
                                         use_tc_tiling_on_sc=False),
    scratch_types=[
        pltpu.VMEM((C, OUT), jnp.float32),
        pltpu.VMEM((C, 8), jnp.float32),
        pltpu.VMEM((C, 16), jnp.float32),
        pltpu.VMEM((C, 16), jnp.float32),
        pltpu.VMEM((C,), jnp.int32),
        pltpu.VMEM((C,), jnp.int32),
        pltpu.VMEM((C,), jnp.int32),
        pltpu.VMEM((64,), jnp.float32),
        pltpu.VMEM_SHARED((BN, OUT), jnp.float32),
        pltpu.VMEM_SHARED((BN, 8), jnp.float32),
        pltpu.SemaphoreType.DMA,
        pltpu.SemaphoreType.DMA,
        pltpu.SemaphoreType.DMA,
        pltpu.SemaphoreType.DMA,
        pltpu.SemaphoreType.DMA,
    ],
)(_sc_body)


def _post_body(zp_ref, sp_ref, g_ref, b_ref, out_ref):
    z = zp_ref[0] + zp_ref[1]
    ssum = sp_ref[0] + sp_ref[1]
    rec = 1.0 / (ssum + 1e-8)
    trow = lax.broadcasted_iota(jnp.int32, (8, OUT), 0) - 4
    jcol = lax.broadcasted_iota(jnp.int32, (8, OUT), 1) // DH
    p = jnp.where(trow == jcol, 1.0, 0.0)
    z = z * jnp.dot(rec, p, preferred_element_type=jnp.float32)
    z = jnp.where(z > 0, z, jnp.exp(jnp.minimum(z, 0.0)) - 1.0)
    mean = jnp.mean(z, axis=-1, keepdims=True)
    var = jnp.mean((z - mean) ** 2, axis=-1, keepdims=True)
    zn = (z - mean) * lax.rsqrt(var + 1e-5)
    out_ref[...] = zn * g_ref[...] + b_ref[...]


_post_call = pl.pallas_call(
    _post_body,
    grid=(BN // BLK,),
    in_specs=[
        pl.BlockSpec((NC, BLK, OUT), lambda i: (0, i, 0)),
        pl.BlockSpec((NC, BLK, 8), lambda i: (0, i, 0)),
        pl.BlockSpec((1, OUT), lambda i: (0, 0)),
        pl.BlockSpec((1, OUT), lambda i: (0, 0)),
    ],
    out_specs=pl.BlockSpec((BLK, OUT), lambda i: (i, 0)),
    out_shape=jax.ShapeDtypeStruct((BN, OUT), jnp.float32),
)


def kernel(x, emb, edge_index, Wq, Wk, Wv, attn_vec, gamma, beta):
    aqf = attn_vec[:, :DH].reshape(128, 1)
    akf = attn_vec[:, DH:].reshape(128, 1)
    sqk, v, eb = _pre_call(x, emb, Wq, Wk, Wv, aqf, akf)
    eb64 = jnp.broadcast_to(eb.reshape(4, 1), (4, 16)).reshape(64)
    zp, sp = _sc_call(sqk, edge_index[0], edge_index[1], v, eb64)
    return _post_call(zp, sp, gamma.reshape(1, OUT), beta.reshape(1, OUT))
